# baseline (speedup 1.0000x reference)
import functools

import jax
import jax.numpy as jnp
from jax import lax
from jax.experimental import pallas as pl
from jax.experimental.pallas import tpu as pltpu
from jax.experimental.pallas import tpu_sc as plsc

_B, _Q, _C, _T = 8, 900, 91, 25
_ALPHA = 0.25
_N = _B * _T
_NPAD = 208
_NVEC = _NPAD // 16


def _ce_body(logits_ref, src_col_ref, src_row_ref, lab_row_ref, out_ref):
    b = pl.program_id(0)
    x = logits_ref[0]
    src_c = src_col_ref[0]
    src_r = src_row_ref[0]
    lab_r = lab_row_ref[0]

    ti = lax.broadcasted_iota(jnp.int32, (_T, _T), 0)
    tj = lax.broadcasted_iota(jnp.int32, (_T, _T), 1)
    dup_later = (src_c == src_r) & (ti > tj)
    wins_row = jnp.logical_not(jnp.any(dup_later, axis=0, keepdims=True))

    qi = lax.broadcasted_iota(jnp.int32, (_Q, _T), 0)
    m = (qi == src_r) & wins_row
    tcq = _C + jnp.sum(jnp.where(m, lab_r - _C, 0), axis=1, keepdims=True)

    ci = lax.broadcasted_iota(jnp.int32, (_Q, _C), 1)
    ispos = ci == tcq

    e = jnp.exp(-jnp.abs(x))
    onep = 1.0 + e
    sp = jnp.maximum(x, 0.0) + jnp.log(onep)
    sig = jnp.where(x >= 0.0, 1.0, e) / onep
    neg = (1.0 - _ALPHA) * sig * sig * sp
    pos = _ALPHA * (1.0 - sig) * (1.0 - sig) * (sp - x)
    partial = jnp.sum(jnp.where(ispos, pos, neg), axis=0, keepdims=True)

    @pl.when(b == 0)
    def _init():
        out_ref[...] = jnp.zeros((1, _C), jnp.float32)

    out_ref[...] += partial


def _ce_call(pred_logits, src_i, lab_i):
    return pl.pallas_call(
        _ce_body,
        grid=(_B,),
        in_specs=[
            pl.BlockSpec((1, _Q, _C), lambda b: (b, 0, 0)),
            pl.BlockSpec((1, _T, 1), lambda b: (b, 0, 0)),
            pl.BlockSpec((1, 1, _T), lambda b: (b, 0, 0)),
            pl.BlockSpec((1, 1, _T), lambda b: (b, 0, 0)),
        ],
        out_specs=pl.BlockSpec((1, _C), lambda b: (0, 0)),
        out_shape=jax.ShapeDtypeStruct((1, _C), jnp.float32),
    )(pred_logits, src_i[:, :, None], src_i[:, None, :], lab_i[:, None, :])


def _box_body(pred_hbm, tgt_hbm, gidx_hbm, out_hbm, pred_v, tgt_v, idx_v, out_v):
    wid = lax.axis_index("s") * 2 + lax.axis_index("c")

    @pl.when(wid == 0)
    def _():
        pltpu.sync_copy(pred_hbm, pred_v)
        pltpu.sync_copy(tgt_hbm, tgt_v)
        pltpu.sync_copy(gidx_hbm, idx_v)
        iot = lax.broadcasted_iota(jnp.int32, (16,), 0)
        l1_acc = jnp.zeros((16,), jnp.float32)
        gi_acc = jnp.zeros((16,), jnp.float32)
        for i in range(_NVEC):
            rows = idx_v[pl.ds(i * 16, 16)]
            sof = rows * 4
            tof = (iot + (i * 16)) * 4

            def _g(ref, base, c):
                return plsc.load_gather(ref, [base + c])

            scx = _g(pred_v, sof, 0)
            scy = _g(pred_v, sof, 1)
            sw = _g(pred_v, sof, 2)
            sh = _g(pred_v, sof, 3)
            tcx = _g(tgt_v, tof, 0)
            tcy = _g(tgt_v, tof, 1)
            tw = _g(tgt_v, tof, 2)
            th = _g(tgt_v, tof, 3)

            l1 = (jnp.abs(scx - tcx) + jnp.abs(scy - tcy)
                  + jnp.abs(sw - tw) + jnp.abs(sh - th))

            sx0 = scx - 0.5 * sw
            sy0 = scy - 0.5 * sh
            sx1 = scx + 0.5 * sw
            sy1 = scy + 0.5 * sh
            tx0 = tcx - 0.5 * tw
            ty0 = tcy - 0.5 * th
            tx1 = tcx + 0.5 * tw
            ty1 = tcy + 0.5 * th

            area1 = (sx1 - sx0) * (sy1 - sy0)
            area2 = (tx1 - tx0) * (ty1 - ty0)
            wi = jnp.maximum(jnp.minimum(sx1, tx1) - jnp.maximum(sx0, tx0), 0.0)
            hi = jnp.maximum(jnp.minimum(sy1, ty1) - jnp.maximum(sy0, ty0), 0.0)
            inter = wi * hi
            union = area1 + area2 - inter
            iou = inter / union
            we = jnp.maximum(jnp.maximum(sx1, tx1) - jnp.minimum(sx0, tx0), 0.0)
            he = jnp.maximum(jnp.maximum(sy1, ty1) - jnp.minimum(sy0, ty0), 0.0)
            areae = we * he
            giou = iou - (areae - union) / areae

            if i == _NVEC - 1:
                valid = (iot + (i * 16)) < _N
                l1 = jnp.where(valid, l1, 0.0)
                one_m_giou = jnp.where(valid, 1.0 - giou, 0.0)
            else:
                one_m_giou = 1.0 - giou
            l1_acc = l1_acc + l1
            gi_acc = gi_acc + one_m_giou
        out_v[pl.ds(0, 16)] = l1_acc
        out_v[pl.ds(16, 16)] = gi_acc
        pltpu.sync_copy(out_v, out_hbm)


@functools.cache
def _get_box_call():
    mesh = plsc.VectorSubcoreMesh(core_axis_name="c", subcore_axis_name="s")
    return pl.kernel(
        _box_body,
        mesh=mesh,
        compiler_params=pltpu.CompilerParams(needs_layout_passes=False),
        out_type=jax.ShapeDtypeStruct((32,), jnp.float32),
        scratch_types=[
            pltpu.VMEM((_B * _Q * 4,), jnp.float32),
            pltpu.VMEM((_NPAD * 4,), jnp.float32),
            pltpu.VMEM((_NPAD,), jnp.int32),
            pltpu.VMEM((32,), jnp.float32),
        ],
    )


def kernel(pred_logits, pred_boxes, tgt_boxes, tgt_labels, src_idx):
    src_i = src_idx.astype(jnp.int32)
    lab_i = tgt_labels.astype(jnp.int32)

    ce = _ce_call(pred_logits, src_i, lab_i)

    gidx = (jnp.arange(_B, dtype=jnp.int32)[:, None] * _Q + src_i).reshape(-1)
    gidx = jnp.pad(gidx, (0, _NPAD - _N))
    tgt_flat = jnp.pad(tgt_boxes.reshape(-1), (0, (_NPAD - _N) * 4))
    box = _get_box_call()(pred_boxes.reshape(-1), tgt_flat, gidx)

    nb = jnp.float32(_N)
    return jnp.stack([
        jnp.sum(ce) / nb,
        jnp.sum(box[:16]) / nb,
        jnp.sum(box[16:]) / nb,
    ])
